# Initial kernel scaffold; baseline (speedup 1.0000x reference)
import functools

import jax
import jax.numpy as jnp
from jax import lax
from jax.experimental import pallas as pl
from jax.experimental.pallas import tpu as pltpu
from jax.experimental.pallas import tpu_sc as plsc

N = 10000
E = 320000
D = 128
NUM_CORES = 2
NUM_SUBCORES = 16
NUM_WORKERS = NUM_CORES * NUM_SUBCORES
EPW = E // NUM_WORKERS
CHUNK = 80
NCHUNKS = EPW // CHUNK
ROWS_PER_TILE = N // NUM_SUBCORES
ZROWS = 125
BLK_N = 400


def _mm_tanh_body(x_ref, w_ref, o_ref):
    o_ref[...] = jnp.tanh(
        jnp.dot(x_ref[...], w_ref[...], preferred_element_type=jnp.float32)
    )


def _mm_tanh(x, w):
    return pl.pallas_call(
        _mm_tanh_body,
        grid=(N // BLK_N,),
        in_specs=[
            pl.BlockSpec((BLK_N, D), lambda i: (i, 0)),
            pl.BlockSpec((D, D), lambda i: (0, 0)),
        ],
        out_specs=pl.BlockSpec((BLK_N, D), lambda i: (i, 0)),
        out_shape=jax.ShapeDtypeStruct((N, D), jnp.float32),
    )(x, w)


def _combine_body(p_ref, o_ref):
    o_ref[...] = p_ref[0] + p_ref[1]


def _combine(p):
    return pl.pallas_call(
        _combine_body,
        grid=(N // BLK_N,),
        in_specs=[pl.BlockSpec((NUM_CORES, BLK_N, D), lambda i: (0, i, 0))],
        out_specs=pl.BlockSpec((BLK_N, D), lambda i: (i, 0)),
        out_shape=jax.ShapeDtypeStruct((N, D), jnp.float32),
    )(p)


def _combine_bias_body(p_ref, b_ref, o_ref):
    o_ref[...] = p_ref[0] + p_ref[1] + b_ref[...]


def _combine_bias(p, b2d):
    return pl.pallas_call(
        _combine_bias_body,
        grid=(N // BLK_N,),
        in_specs=[
            pl.BlockSpec((NUM_CORES, BLK_N, D), lambda i: (0, i, 0)),
            pl.BlockSpec((1, D), lambda i: (0, 0)),
        ],
        out_specs=pl.BlockSpec((BLK_N, D), lambda i: (i, 0)),
        out_shape=jax.ShapeDtypeStruct((N, D), jnp.float32),
    )(p, b2d)


def _spmm_body(x_hbm, src_hbm, dst_hbm, val_hbm, out_hbm,
               src_v, dst_v, val_v, rows_v, zbuf, acc_sh, sem):
    cid = lax.axis_index("c")
    sid = lax.axis_index("s")
    wid = sid * NUM_CORES + cid
    base_e = wid * EPW
    row0 = sid * ROWS_PER_TILE

    def zrow(i, c):
        for r in range(D // 16):
            zbuf[i, pl.ds(r * 16, 16)] = jnp.zeros((16,), jnp.float32)
        return c
    lax.fori_loop(0, ZROWS, zrow, 0)
    for j in range(ROWS_PER_TILE // ZROWS):
        pltpu.sync_copy(zbuf, acc_sh.at[pl.ds(row0 + j * ZROWS, ZROWS)])
    plsc.subcore_barrier()

    def chunk_body(k, c):
        eoff = base_e + k * CHUNK
        pltpu.sync_copy(src_hbm.at[pl.ds(eoff, CHUNK)], src_v)
        pltpu.sync_copy(dst_hbm.at[pl.ds(eoff, CHUNK)], dst_v)
        pltpu.sync_copy(val_hbm.at[pl.ds(eoff, CHUNK)], val_v)
        pltpu.async_copy(x_hbm.at[src_v], rows_v, sem).wait()

        def edge_body(e, c2):
            v = val_v[e]
            for r in range(D // 16):
                sl = pl.ds(r * 16, 16)
                rows_v[e, sl] = rows_v[e, sl] * v
            return c2
        lax.fori_loop(0, CHUNK, edge_body, 0)

        pltpu.sync_copy(rows_v, acc_sh.at[dst_v], add=True)
        return c
    lax.fori_loop(0, NCHUNKS, chunk_body, 0)

    plsc.subcore_barrier()
    for j in range(ROWS_PER_TILE // ZROWS):
        r0 = row0 + j * ZROWS
        pltpu.sync_copy(acc_sh.at[pl.ds(r0, ZROWS)],
                        out_hbm.at[cid].at[pl.ds(r0, ZROWS)])


_spmm = pl.kernel(
    _spmm_body,
    out_type=jax.ShapeDtypeStruct((NUM_CORES, N, D), jnp.float32),
    mesh=plsc.VectorSubcoreMesh(core_axis_name="c", subcore_axis_name="s"),
    scratch_types=[
        pltpu.VMEM((CHUNK,), jnp.int32),
        pltpu.VMEM((CHUNK,), jnp.int32),
        pltpu.VMEM((CHUNK,), jnp.float32),
        pltpu.VMEM((CHUNK, D), jnp.float32),
        pltpu.VMEM((ZROWS, D), jnp.float32),
        pltpu.VMEM_SHARED((N, D), jnp.float32),
        pltpu.SemaphoreType.DMA,
    ],
)


@jax.jit
def kernel(features, adj_indices, adj_values, W, b):
    dst = adj_indices[0]
    src = adj_indices[1]
    h = _mm_tanh(features, W)
    p1 = _spmm(h, src, dst, adj_values)
    h1 = _combine(p1)
    p2 = _spmm(h1, src, dst, adj_values)
    return _combine_bias(p2, b.reshape(1, D))


# speedup vs baseline: 9.1743x; 9.1743x over previous
import jax
import jax.numpy as jnp
from jax import lax
from jax.experimental import pallas as pl
from jax.experimental.pallas import tpu as pltpu
from jax.experimental.pallas import tpu_sc as plsc

N = 10000
E = 320000
D = 128
NUM_CORES = 2
NUM_SUBCORES = 16
NUM_WORKERS = NUM_CORES * NUM_SUBCORES
EPW = E // NUM_WORKERS
CHUNK = 80
NCHUNKS = EPW // CHUNK
NP = 10240
ROWS_PER_TILE = NP // NUM_SUBCORES
BLK_N = 400
NBUF = 2


def _mm_tanh_body(x_ref, w_ref, o_ref):
    o_ref[...] = jnp.tanh(
        jnp.dot(x_ref[...], w_ref[...], preferred_element_type=jnp.float32)
    )


def _mm_tanh(x, w):
    return pl.pallas_call(
        _mm_tanh_body,
        grid=(N // BLK_N,),
        in_specs=[
            pl.BlockSpec((BLK_N, D), lambda i: (i, 0)),
            pl.BlockSpec((D, D), lambda i: (0, 0)),
        ],
        out_specs=pl.BlockSpec((BLK_N, D), lambda i: (i, 0)),
        out_shape=jax.ShapeDtypeStruct((N, D), jnp.float32),
    )(x, w)


def _combine_body(p_ref, o_ref):
    o_ref[...] = p_ref[0] + p_ref[1]


def _combine(p):
    return pl.pallas_call(
        _combine_body,
        grid=(N // BLK_N,),
        in_specs=[pl.BlockSpec((NUM_CORES, BLK_N, D), lambda i: (0, i, 0))],
        out_specs=pl.BlockSpec((BLK_N, D), lambda i: (i, 0)),
        out_shape=jax.ShapeDtypeStruct((N, D), jnp.float32),
    )(p)


def _combine_bias_body(p_ref, b_ref, o_ref):
    o_ref[...] = p_ref[0] + p_ref[1] + b_ref[...]


def _combine_bias(p, b2d):
    return pl.pallas_call(
        _combine_bias_body,
        grid=(N // BLK_N,),
        in_specs=[
            pl.BlockSpec((NUM_CORES, BLK_N, D), lambda i: (0, i, 0)),
            pl.BlockSpec((1, D), lambda i: (0, 0)),
        ],
        out_specs=pl.BlockSpec((BLK_N, D), lambda i: (i, 0)),
        out_shape=jax.ShapeDtypeStruct((N, D), jnp.float32),
    )(p, b2d)


def _unpack_src(packed_v, k, dst_ref):
    sh = jnp.full((16,), 16, jnp.int32)
    for g in range(CHUNK // 16):
        p16 = packed_v[pl.ds(k * CHUNK + g * 16, 16)]
        dst_ref[pl.ds(g * 16, 16)] = lax.shift_right_logical(p16, sh)


def _unpack_dst(packed_v, k, dst_ref):
    mask = jnp.full((16,), 0xFFFF, jnp.int32)
    for g in range(CHUNK // 16):
        p16 = packed_v[pl.ds(k * CHUNK + g * 16, 16)]
        dst_ref[pl.ds(g * 16, 16)] = lax.bitwise_and(p16, mask)


def _spmm_body(x_hbm, packed_hbm, vals_hbm, out_hbm,
               packed_v, vals_v,
               r0b, r1b, si0, si1, di0,
               acc_sh, s0, s1):
    rows = [r0b, r1b]
    sidx = [si0, si1]
    sems = [s0, s1]
    cid = lax.axis_index("c")
    sid = lax.axis_index("s")
    wid = sid * NUM_CORES + cid
    row0 = sid * ROWS_PER_TILE

    pltpu.sync_copy(packed_hbm.at[wid], packed_v)
    pltpu.sync_copy(vals_hbm.at[wid], vals_v)

    def zrow(i, c):
        for r in range(D // 16):
            r0b[i, pl.ds(r * 16, 16)] = jnp.zeros((16,), jnp.float32)
        return c
    lax.fori_loop(0, CHUNK, zrow, 0)
    for j in range(ROWS_PER_TILE // CHUNK):
        pltpu.sync_copy(r0b, acc_sh.at[pl.ds(row0 + j * CHUNK, CHUNK)])

    for b in range(NBUF):
        _unpack_src(packed_v, b, sidx[b])
        pltpu.async_copy(x_hbm.at[sidx[b]], rows[b], sems[b])

    plsc.subcore_barrier()

    def chunk_step(k, b, refill=True):
        pltpu.make_async_copy(x_hbm.at[sidx[b]], rows[b], sems[b]).wait()
        _unpack_dst(packed_v, k, di0)

        def sgrp(g2, c2, _rows=rows[b], _k=k):
            vv = vals_v[pl.ds(_k * CHUNK + g2 * 16, 16)]
            for j in range(16):
                e = g2 * 16 + j
                v = vv[j]
                for r in range(D // 16):
                    sl = pl.ds(r * 16, 16)
                    _rows[e, sl] = _rows[e, sl] * v
            return c2
        lax.fori_loop(0, CHUNK // 16, sgrp, 0)

        pltpu.sync_copy(rows[b], acc_sh.at[di0], add=True)

        if refill:
            kn = k + NBUF

            @pl.when(kn < NCHUNKS)
            def _():
                _unpack_src(packed_v, kn, sidx[b])
                pltpu.async_copy(x_hbm.at[sidx[b]], rows[b], sems[b])

    def giter(g, c):
        for b in range(NBUF):
            chunk_step(g * NBUF + b, b)
        return c
    lax.fori_loop(0, (NCHUNKS - 1) // NBUF, giter, 0)
    chunk_step(NCHUNKS - 1, (NCHUNKS - 1) % NBUF, refill=False)

    plsc.subcore_barrier()
    pltpu.sync_copy(acc_sh.at[pl.ds(row0, ROWS_PER_TILE)],
                    out_hbm.at[cid, pl.ds(row0, ROWS_PER_TILE)])


_spmm = pl.kernel(
    _spmm_body,
    out_type=jax.ShapeDtypeStruct((NUM_CORES, NP, D), jnp.float32),
    mesh=plsc.VectorSubcoreMesh(core_axis_name="c", subcore_axis_name="s"),
    scratch_types=[
        pltpu.VMEM((EPW,), jnp.int32),
        pltpu.VMEM((EPW,), jnp.float32),
        pltpu.VMEM((CHUNK, D), jnp.float32),
        pltpu.VMEM((CHUNK, D), jnp.float32),
        pltpu.VMEM((CHUNK,), jnp.int32),
        pltpu.VMEM((CHUNK,), jnp.int32),
        pltpu.VMEM((CHUNK,), jnp.int32),
        pltpu.VMEM_SHARED((NP, D), jnp.float32),
        pltpu.SemaphoreType.DMA,
        pltpu.SemaphoreType.DMA,
    ],
)


@jax.jit
def kernel(features, adj_indices, adj_values, W, b):
    dst = adj_indices[0]
    src = adj_indices[1]
    packed = (src * 65536 + dst).reshape(NUM_WORKERS, EPW)
    vals = adj_values.reshape(NUM_WORKERS, EPW)
    h = _mm_tanh(features, W)
    p1 = _spmm(h, packed, vals)
    h1 = _combine(p1)
    p2 = _spmm(h1, packed, vals)
    return _combine_bias(p2, b.reshape(1, D))
